# baseline (speedup 1.0000x reference)
import jax
import jax.numpy as jnp
from jax.experimental import pallas as pl
from jax.experimental.pallas import tpu as pltpu

_P = 32
_C = 64
_NB = 256
_NB2 = 256


def _trunk_kernel(xin_ref, ids8_ref, bfirst_ref, bias0_ref, eids_ref,
                  B_ref, bias_ref, bd_ref, lb_ref,
                  vsel_ref, vb_ref, u1y_ref, u2T_ref, u2b_ref,
                  feat0_ref, val_ref,
                  act_ref, stagein_ref):
    j = pl.program_id(1)
    f32 = jnp.float32

    @pl.when(j == 0)
    def _prologue():
        xin = xin_ref[...]
        ids512 = jax.lax.dot_general(
            ids8_ref[...], eids_ref[...], (((2,), (0,)), ((), ())),
            preferred_element_type=f32)
        rows = []
        for y in range(8):
            acc = jnp.zeros((_NB, 8 * _C), f32)
            for ky in range(7):
                yi = y + ky - 3
                if 0 <= yi < 8:
                    acc = acc + jnp.dot(xin[:, yi, :], bfirst_ref[ky],
                                        preferred_element_type=f32)
            rows.append(jax.nn.relu(acc + bias0_ref[0]))
        feat = jnp.stack(rows, axis=1) + ids512
        act_ref[...] = feat
        feat0_ref[...] = feat

    @pl.when((j >= 1) & (j <= 49))
    def _conv_step():
        jj = (j - 1) % 7

        @pl.when(jj == 0)
        def _save_stage_input():
            stagein_ref[...] = act_ref[...]

        act = act_ref[...]
        bias = bias_ref[0, 0]
        rows = []
        for y in range(8):
            acc = jnp.zeros((_NB, 8 * _C), f32)
            for ky in range(7):
                yi = y + ky - 3
                if 0 <= yi < 8:
                    acc = acc + jnp.dot(act[:, yi, :], B_ref[0, ky],
                                        preferred_element_type=f32)
            rows.append(jax.nn.relu(acc + bias))
        new = jnp.stack(rows, axis=1)

        @pl.when(jj != 6)
        def _plain():
            act_ref[...] = new

        @pl.when(jj == 6)
        def _stage_end():
            t = jax.lax.dot_general(new, bd_ref[0], (((2,), (0,)), ((), ())),
                                    preferred_element_type=f32)
            t = t + lb_ref[0, 0]
            act_ref[...] = jax.nn.relu(t + stagein_ref[...])

    @pl.when(j == 50)
    def _epilogue():
        feat = act_ref[...]
        vb64 = jax.lax.dot_general(feat, vsel_ref[...],
                                   (((2,), (0,)), ((), ())),
                                   preferred_element_type=f32)
        vb64 = jax.nn.relu(vb64 + vb_ref[0, 0])
        vacc = jnp.zeros((_NB, 256), f32)
        for y in range(8):
            vacc = vacc + jnp.dot(vb64[:, y, :], u1y_ref[y],
                                  preferred_element_type=f32)
        v = jax.nn.relu(vacc)
        val_ref[...] = jnp.tanh(jnp.dot(v, u2T_ref[...],
                                        preferred_element_type=f32)
                                + u2b_ref[0])


def _heads_kernel(feat_ref, ids_ref, w1p_ref, b1_ref, w2T_ref, b2_ref,
                  pol_ref, pid_ref):
    f32 = jnp.float32
    ids_i = ids_ref[...].astype(jnp.int32)
    pcls = jax.lax.broadcasted_iota(jnp.int32, (1, _P, 1), 1) + 1
    match = ids_i[:, None, :] == pcls
    pos = jax.lax.broadcasted_iota(jnp.int32, (1, 1, 64), 2)
    first = jnp.min(jnp.where(match, pos, 64), axis=-1)
    oh = (pos == first[..., None]).astype(f32)
    pv = jax.lax.dot_general(oh, feat_ref[...],
                             (((2,), (1,)), ((0,), (0,))),
                             preferred_element_type=f32)
    hid = jnp.zeros((_NB2, 256), f32)
    for p in range(_P):
        hid = hid + jnp.dot(pv[:, p, :], w1p_ref[p],
                            preferred_element_type=f32)
    hid = jax.nn.relu(hid + b1_ref[0])
    pol_ref[...] = jnp.dot(hid, w2T_ref[...],
                           preferred_element_type=f32) + b2_ref[0]

    present = (first < 64).astype(f32)
    pnum = (jax.lax.broadcasted_iota(jnp.int32, (1, _P), 1) + 1).astype(f32)
    pid = present * pnum
    pid_ref[...] = jnp.broadcast_to(pid[:, None, :], (_NB2, 8, _P))


def kernel(x, first_w, first_s, first_b, res_w, res_s, res_b,
           last_w, last_s, last_b, pfc1_w, pfc1_b, pfc2_w, pfc2_b,
           vconv_w, v_s, v_b, vfc1_w, vfc1_b, vfc2_w, vfc2_b):
    N = x.shape[0]
    S, NBK = res_w.shape[0], res_w.shape[1]
    L = S * NBK
    C, P = _C, _P
    f32 = jnp.float32

    xin = x[:, :-1].transpose(0, 2, 3, 1).reshape(N, 8, 8 * 12)
    ids8 = x[:, -1]
    ids64 = x[:, -1].reshape(N, 64)

    xi = jnp.arange(8)
    dxm = xi[:, None] - xi[None, :] + 3
    S_sel = (jax.lax.broadcasted_iota(jnp.int32, (7, 8, 8), 0)
             == dxm[None, :, :]).astype(f32)

    w0 = first_w * first_s[:, None, None, None]
    Bf = jnp.einsum('ocyk,kab->yacbo', w0, S_sel)
    Bf = jnp.pad(Bf, ((0, 0), (0, 0), (0, 0), (0, 0), (0, 1)))
    Bf = Bf.reshape(7, 8 * 12, 8 * C)
    bias0 = jnp.tile(jnp.pad(first_b, (0, 1)), 8).reshape(1, 8 * C)
    eids = (jnp.arange(8 * C)[None, :]
            == (jnp.arange(8) * C + (C - 1))[:, None]).astype(f32)

    wr = (res_w * res_s[:, :, :, None, None, None]).reshape(L, C, C, 7, 7)
    B = jnp.einsum('locyk,kab->lyacbo', wr, S_sel)
    B = B.reshape(L, 7, 8 * C, 8 * C)
    bias = jnp.tile(res_b.reshape(L, C), (1, 8)).reshape(L, 1, 8 * C)

    lwT = (last_w * last_s[:, :, None]).transpose(0, 2, 1)
    BD = jnp.einsum('ico,xy->ixcyo', lwT, jnp.eye(8, dtype=f32))
    BD = BD.reshape(S, 8 * C, 8 * C)
    lb = jnp.tile(last_b, (1, 8)).reshape(S, 1, 8 * C)

    vwv = vconv_w.reshape(C) * v_s[0]
    vsel = ((jnp.arange(8 * C)[:, None] // C == jnp.arange(8)[None, :])
            .astype(f32) * vwv[jnp.tile(jnp.arange(C), 8)][:, None])
    vb = v_b.reshape(1, 1)
    u1y = vfc1_w.T.reshape(8, 8, 256)
    u2T = vfc2_w.T
    u2b = vfc2_b.reshape(1, -1)

    nblk = N // _NB
    grid = (nblk, L + 2)

    def _w_idx(nb, j):
        return (jnp.clip(j - 1, 0, L - 1), 0, 0, 0)

    def _w3_idx(nb, j):
        return (jnp.clip(j - 1, 0, L - 1), 0, 0)

    def _s_idx(nb, j):
        return (jnp.clip((j - 1) // NBK, 0, S - 1), 0, 0)

    full2 = lambda nb, j: (0, 0)
    full3 = lambda nb, j: (0, 0, 0)

    feat0, value = pl.pallas_call(
        _trunk_kernel,
        grid=grid,
        in_specs=[
            pl.BlockSpec((_NB, 8, 96), lambda nb, j: (nb, 0, 0)),
            pl.BlockSpec((_NB, 8, 8), lambda nb, j: (nb, 0, 0)),
            pl.BlockSpec((7, 96, 8 * C), full3),
            pl.BlockSpec((1, 8 * C), full2),
            pl.BlockSpec((8, 8 * C), full2),
            pl.BlockSpec((1, 7, 8 * C, 8 * C), _w_idx),
            pl.BlockSpec((1, 1, 8 * C), _w3_idx),
            pl.BlockSpec((1, 8 * C, 8 * C), _s_idx),
            pl.BlockSpec((1, 1, 8 * C), _s_idx),
            pl.BlockSpec((8 * C, 8), full2),
            pl.BlockSpec((1, 1), full2),
            pl.BlockSpec((8, 8, 256), full3),
            pl.BlockSpec((256, 1), full2),
            pl.BlockSpec((1, 1), full2),
        ],
        out_specs=[
            pl.BlockSpec((_NB, 8, 8 * C), lambda nb, j: (nb, 0, 0)),
            pl.BlockSpec((_NB, 1), lambda nb, j: (nb, 0)),
        ],
        out_shape=[
            jax.ShapeDtypeStruct((N, 8, 8 * C), f32),
            jax.ShapeDtypeStruct((N, 1), f32),
        ],
        scratch_shapes=[
            pltpu.VMEM((_NB, 8, 8 * C), f32),
            pltpu.VMEM((_NB, 8, 8 * C), f32),
        ],
        compiler_params=pltpu.CompilerParams(
            dimension_semantics=("parallel", "arbitrary"),
        ),
    )(xin, ids8, Bf, bias0, eids, B, bias, BD, lb,
      vsel, vb, u1y, u2T, u2b)

    board = feat0.reshape(N, 64, C)
    w1p = pfc1_w.T.reshape(P, C, 256)
    b1 = pfc1_b.reshape(1, -1)
    w2T = pfc2_w.T
    b2 = pfc2_b.reshape(1, -1)

    policy, pids = pl.pallas_call(
        _heads_kernel,
        grid=(N // _NB2,),
        in_specs=[
            pl.BlockSpec((_NB2, 64, C), lambda nb: (nb, 0, 0)),
            pl.BlockSpec((_NB2, 64), lambda nb: (nb, 0)),
            pl.BlockSpec((P, C, 256), lambda nb: (0, 0, 0)),
            pl.BlockSpec((1, 256), lambda nb: (0, 0)),
            pl.BlockSpec((256, 1700), lambda nb: (0, 0)),
            pl.BlockSpec((1, 1700), lambda nb: (0, 0)),
        ],
        out_specs=[
            pl.BlockSpec((_NB2, 1700), lambda nb: (nb, 0)),
            pl.BlockSpec((_NB2, 8, P), lambda nb: (nb, 0, 0)),
        ],
        out_shape=[
            jax.ShapeDtypeStruct((N, 1700), f32),
            jax.ShapeDtypeStruct((N, 8, P), f32),
        ],
        compiler_params=pltpu.CompilerParams(
            dimension_semantics=("parallel",),
        ),
    )(board, ids64, w1p, b1, w2T, b2)

    return (policy, value, pids)


# speedup vs baseline: 1.2187x; 1.2187x over previous
import jax
import jax.numpy as jnp
from jax.experimental import pallas as pl
from jax.experimental.pallas import tpu as pltpu

_P = 32
_C = 64
_NB = 256
_NB2 = 256


def _trunk_kernel(xin_ref, ids8_ref, bfirst_ref, bias0_ref, eids_ref,
                  B_ref, bias_ref, bd_ref, lb_ref,
                  feat0_ref, featF_ref,
                  act_ref, new_ref, stagein_ref):
    j = pl.program_id(1)
    f32 = jnp.float32
    R = _NB

    def _banded_accum(src, op_ref, kdim):
        new_ref[...] = jnp.dot(src, op_ref[3], preferred_element_type=f32)
        for s in (-3, -2, -1, 1, 2, 3):
            ao = max(0, -s)
            bo = 8 - max(0, s)
            m = (bo - ao) * R
            ai = (ao + s) * R
            new_ref[pl.ds(ao * R, m), :] += jnp.dot(
                src[ai:ai + m, :], op_ref[s + 3],
                preferred_element_type=f32)

    @pl.when(j == 0)
    def _prologue():
        xin = xin_ref[...].reshape(8 * R, 8 * 12)
        _banded_accum(xin, bfirst_ref, 96)
        ids512 = jnp.dot(ids8_ref[...].reshape(8 * R, 8), eids_ref[...],
                         preferred_element_type=f32)
        feat = jax.nn.relu(new_ref[...] + bias0_ref[0]) + ids512
        act_ref[...] = feat
        feat0_ref[...] = feat.reshape(8, R, 8 * _C)

    @pl.when((j >= 1) & (j <= 49))
    def _conv_step():
        jj = (j - 1) % 7

        @pl.when(jj == 0)
        def _save_stage_input():
            stagein_ref[...] = act_ref[...]

        _banded_accum(act_ref[...], B_ref[0], 8 * _C)
        bias = bias_ref[0, 0]

        @pl.when(jj != 6)
        def _plain():
            act_ref[...] = jax.nn.relu(new_ref[...] + bias)

        @pl.when(jj == 6)
        def _stage_end():
            r7 = jax.nn.relu(new_ref[...] + bias)
            t = jnp.dot(r7, bd_ref[0], preferred_element_type=f32)
            out = jax.nn.relu(t + lb_ref[0, 0] + stagein_ref[...])
            act_ref[...] = out

            @pl.when(j == 49)
            def _export_final():
                featF_ref[...] = out.reshape(8, R, 8 * _C)


def _heads_kernel(board0_ref, ids_ref, boardF_ref, w1p_ref, b1_ref,
                  w2T_ref, b2_ref, vw_ref, vb_ref,
                  u1T_ref, u1b_ref, u2T_ref, u2b_ref,
                  pol_ref, val_ref, pid_ref):
    f32 = jnp.float32
    ids_i = ids_ref[...].astype(jnp.int32)
    pcls = jax.lax.broadcasted_iota(jnp.int32, (1, _P, 1), 1) + 1
    match = ids_i[:, None, :] == pcls
    pos = jax.lax.broadcasted_iota(jnp.int32, (1, 1, 64), 2)
    first = jnp.min(jnp.where(match, pos, 64), axis=-1)
    oh = (pos == first[..., None]).astype(f32)
    pv = jax.lax.dot_general(oh, board0_ref[...],
                             (((2,), (1,)), ((0,), (0,))),
                             preferred_element_type=f32)
    hid = jnp.zeros((_NB2, 256), f32)
    for p in range(_P):
        hid = hid + jnp.dot(pv[:, p, :], w1p_ref[p],
                            preferred_element_type=f32)
    hid = jax.nn.relu(hid + b1_ref[0])
    pol_ref[...] = jnp.dot(hid, w2T_ref[...],
                           preferred_element_type=f32) + b2_ref[0]

    v64 = jnp.sum(boardF_ref[...] * vw_ref[0][None, None, :], axis=-1)
    v64 = jax.nn.relu(v64 + vb_ref[0, 0])
    v = jax.nn.relu(jnp.dot(v64, u1T_ref[...],
                            preferred_element_type=f32) + u1b_ref[0])
    val_ref[...] = jnp.tanh(jnp.dot(v, u2T_ref[...],
                                    preferred_element_type=f32)
                            + u2b_ref[0])

    present = (first < 64).astype(f32)
    pnum = (jax.lax.broadcasted_iota(jnp.int32, (1, _P), 1) + 1).astype(f32)
    pid = present * pnum
    pid_ref[...] = jnp.broadcast_to(pid[:, None, :], (_NB2, 8, _P))


def kernel(x, first_w, first_s, first_b, res_w, res_s, res_b,
           last_w, last_s, last_b, pfc1_w, pfc1_b, pfc2_w, pfc2_b,
           vconv_w, v_s, v_b, vfc1_w, vfc1_b, vfc2_w, vfc2_b):
    N = x.shape[0]
    S, NBK = res_w.shape[0], res_w.shape[1]
    L = S * NBK
    C, P = _C, _P
    f32 = jnp.float32

    xin = x[:, :-1].transpose(2, 0, 3, 1).reshape(8, N, 8 * 12)
    ids8 = x[:, -1].transpose(1, 0, 2)
    ids64 = x[:, -1].reshape(N, 64)

    xi = jnp.arange(8)
    dxm = xi[:, None] - xi[None, :] + 3
    S_sel = (jax.lax.broadcasted_iota(jnp.int32, (7, 8, 8), 0)
             == dxm[None, :, :]).astype(f32)

    w0 = first_w * first_s[:, None, None, None]
    Bf = jnp.einsum('ocyk,kab->yacbo', w0, S_sel)
    Bf = jnp.pad(Bf, ((0, 0), (0, 0), (0, 0), (0, 0), (0, 1)))
    Bf = Bf.reshape(7, 8 * 12, 8 * C)
    bias0 = jnp.tile(jnp.pad(first_b, (0, 1)), 8).reshape(1, 8 * C)
    eids = (jnp.arange(8 * C)[None, :]
            == (jnp.arange(8) * C + (C - 1))[:, None]).astype(f32)

    wr = (res_w * res_s[:, :, :, None, None, None]).reshape(L, C, C, 7, 7)
    B = jnp.einsum('locyk,kab->lyacbo', wr, S_sel)
    B = B.reshape(L, 7, 8 * C, 8 * C)
    bias = jnp.tile(res_b.reshape(L, C), (1, 8)).reshape(L, 1, 8 * C)

    lwT = (last_w * last_s[:, :, None]).transpose(0, 2, 1)
    BD = jnp.einsum('ico,xy->ixcyo', lwT, jnp.eye(8, dtype=f32))
    BD = BD.reshape(S, 8 * C, 8 * C)
    lb = jnp.tile(last_b, (1, 8)).reshape(S, 1, 8 * C)

    nblk = N // _NB
    grid = (nblk, L + 1)

    def _w_idx(nb, j):
        return (jnp.clip(j - 1, 0, L - 1), 0, 0, 0)

    def _w3_idx(nb, j):
        return (jnp.clip(j - 1, 0, L - 1), 0, 0)

    def _s_idx(nb, j):
        return (jnp.clip((j - 1) // NBK, 0, S - 1), 0, 0)

    full2 = lambda nb, j: (0, 0)
    full3 = lambda nb, j: (0, 0, 0)

    feat0, featF = pl.pallas_call(
        _trunk_kernel,
        grid=grid,
        in_specs=[
            pl.BlockSpec((8, _NB, 96), lambda nb, j: (0, nb, 0)),
            pl.BlockSpec((8, _NB, 8), lambda nb, j: (0, nb, 0)),
            pl.BlockSpec((7, 96, 8 * C), full3),
            pl.BlockSpec((1, 8 * C), full2),
            pl.BlockSpec((8, 8 * C), full2),
            pl.BlockSpec((1, 7, 8 * C, 8 * C), _w_idx),
            pl.BlockSpec((1, 1, 8 * C), _w3_idx),
            pl.BlockSpec((1, 8 * C, 8 * C), _s_idx),
            pl.BlockSpec((1, 1, 8 * C), _s_idx),
        ],
        out_specs=[
            pl.BlockSpec((8, _NB, 8 * C), lambda nb, j: (0, nb, 0)),
            pl.BlockSpec((8, _NB, 8 * C), lambda nb, j: (0, nb, 0)),
        ],
        out_shape=[
            jax.ShapeDtypeStruct((8, N, 8 * C), f32),
            jax.ShapeDtypeStruct((8, N, 8 * C), f32),
        ],
        scratch_shapes=[
            pltpu.VMEM((8 * _NB, 8 * C), f32),
            pltpu.VMEM((8 * _NB, 8 * C), f32),
            pltpu.VMEM((8 * _NB, 8 * C), f32),
        ],
        compiler_params=pltpu.CompilerParams(
            dimension_semantics=("parallel", "arbitrary"),
        ),
    )(xin, ids8, Bf, bias0, eids, B, bias, BD, lb)

    board0 = feat0.reshape(8, N, 8, C).transpose(1, 0, 2, 3).reshape(N, 64, C)
    boardF = featF.reshape(8, N, 8, C).transpose(1, 0, 2, 3).reshape(N, 64, C)

    w1p = pfc1_w.T.reshape(P, C, 256)
    b1 = pfc1_b.reshape(1, -1)
    w2T = pfc2_w.T
    b2 = pfc2_b.reshape(1, -1)
    vw = (vconv_w.reshape(C) * v_s[0]).reshape(1, C)
    vb = v_b.reshape(1, 1)
    u1T = vfc1_w.T
    u1b = vfc1_b.reshape(1, -1)
    u2T = vfc2_w.T
    u2b = vfc2_b.reshape(1, -1)

    policy, value, pids = pl.pallas_call(
        _heads_kernel,
        grid=(N // _NB2,),
        in_specs=[
            pl.BlockSpec((_NB2, 64, C), lambda nb: (nb, 0, 0)),
            pl.BlockSpec((_NB2, 64), lambda nb: (nb, 0)),
            pl.BlockSpec((_NB2, 64, C), lambda nb: (nb, 0, 0)),
            pl.BlockSpec((P, C, 256), lambda nb: (0, 0, 0)),
            pl.BlockSpec((1, 256), lambda nb: (0, 0)),
            pl.BlockSpec((256, 1700), lambda nb: (0, 0)),
            pl.BlockSpec((1, 1700), lambda nb: (0, 0)),
            pl.BlockSpec((1, C), lambda nb: (0, 0)),
            pl.BlockSpec((1, 1), lambda nb: (0, 0)),
            pl.BlockSpec((C, 256), lambda nb: (0, 0)),
            pl.BlockSpec((1, 256), lambda nb: (0, 0)),
            pl.BlockSpec((256, 1), lambda nb: (0, 0)),
            pl.BlockSpec((1, 1), lambda nb: (0, 0)),
        ],
        out_specs=[
            pl.BlockSpec((_NB2, 1700), lambda nb: (nb, 0)),
            pl.BlockSpec((_NB2, 1), lambda nb: (nb, 0)),
            pl.BlockSpec((_NB2, 8, P), lambda nb: (nb, 0, 0)),
        ],
        out_shape=[
            jax.ShapeDtypeStruct((N, 1700), f32),
            jax.ShapeDtypeStruct((N, 1), f32),
            jax.ShapeDtypeStruct((N, 8, P), f32),
        ],
        compiler_params=pltpu.CompilerParams(
            dimension_semantics=("parallel",),
        ),
    )(board0, ids64, boardF, w1p, b1, w2T, b2, vw, vb, u1T, u1b, u2T, u2b)

    return (policy, value, pids)
